# baseline (speedup 1.0000x reference)
import functools

import jax
import jax.numpy as jnp
from jax import lax
from jax.experimental import pallas as pl
from jax.experimental.pallas import tpu as pltpu
from jax.experimental.pallas import tpu_sc as plsc

NUM_USERS = 100000
EMB_DIM = 32
NUM_EDGES = 1600000

_NC = 2
_NS = 16
_NW = _NC * _NS

_HIST = 100352
_HSLICE = _HIST // _NS
_PAD_BIN = _HIST - 1

_K = 128
_ROWS = NUM_EDGES // _K
_TSTEPS = 392
_ROWS_OUT = _NW * _TSTEPS
_ROWS_ALL = _ROWS_OUT + 68

_HROWS = 12608
_HPT = _HROWS // _NS
_HC = 4
_HITER = _HPT // _HC


def _sc_body(user_hbm, src2d, dst2d, gb_hbm, out_hbm,
             hist, zbuf, ones4, hidx, sidx, didx, srows, drows,
             degb, outb, gbv,
             semA0, semA1, semB0, semB1, semO0, semO1, semH0, semH1):
    cid = lax.axis_index("c")
    sid = lax.axis_index("s")
    wid = sid * _NC + cid
    semA = (semA0, semA1)
    semB = (semB0, semB1)
    semO = (semO0, semO1)
    semH = (semH0, semH1)

    def _zero(i, carry):
        zbuf[pl.ds(i * 16, 16)] = jnp.zeros((16,), jnp.float32)
        return carry
    lax.fori_loop(0, _HSLICE // 16, _zero, 0)
    for r in range(_HC):
        for i in range(_K // 16):
            ones4[r, pl.ds(i * 16, 16)] = jnp.ones((16,), jnp.float32)
    pltpu.sync_copy(zbuf, hist.at[pl.ds(pl.multiple_of(sid * _HSLICE, 8),
                                        _HSLICE)])
    plsc.subcore_barrier()

    hrow0 = sid * _HPT

    def _h_load(k, b):
        pltpu.make_async_copy(dst2d.at[pl.ds(hrow0 + k * _HC, _HC), :],
                              hidx.at[b], semH[b]).start()

    def _h_wait(b):
        pltpu.make_async_copy(dst2d.at[pl.ds(0, _HC), :],
                              hidx.at[b], semH[b]).wait()

    def _h_scatter(b):
        for j in range(_HC):
            pltpu.sync_copy(ones4.at[0], hist.at[hidx.at[b, j]], add=True)

    _h_load(0, 0)
    _h_wait(0)
    _h_load(1, 1)
    _h_scatter(0)

    def _h_pair(g, carry):
        k = 2 * g + 1
        _h_wait(1)
        _h_load(k + 1, 0)
        _h_scatter(1)
        _h_wait(0)
        _h_load(k + 2, 1)
        _h_scatter(0)
        return carry
    lax.fori_loop(0, (_HITER - 1) // 2, _h_pair, 0)
    _h_wait(1)
    plsc.subcore_barrier()

    pltpu.sync_copy(gb_hbm, gbv)

    def _issue_idx(t, b):
        c = wid + t * _NW
        pltpu.make_async_copy(src2d.at[c], sidx.at[b], semA[b]).start()
        pltpu.make_async_copy(dst2d.at[c], didx.at[b], semA[b]).start()

    def _wait_idx(b):
        pltpu.make_async_copy(src2d.at[0], sidx.at[b], semA[b]).wait()
        pltpu.make_async_copy(dst2d.at[0], didx.at[b], semA[b]).wait()

    def _issue_gathers(b):
        pltpu.make_async_copy(user_hbm.at[sidx.at[b]], srows.at[b],
                              semB[b]).start()
        pltpu.make_async_copy(user_hbm.at[didx.at[b]], drows.at[b],
                              semB[b]).start()

    def _wait_gathers(b):
        pltpu.make_async_copy(user_hbm.at[pl.ds(0, _K), :], srows.at[b],
                              semB[b]).wait()
        pltpu.make_async_copy(user_hbm.at[pl.ds(0, _K), :], drows.at[b],
                              semB[b]).wait()

    def _issue_out(t, b):
        c = wid + t * _NW
        pltpu.make_async_copy(outb.at[b], out_hbm.at[c], semO[b]).start()

    def _wait_out(b):
        pltpu.make_async_copy(outb.at[b], out_hbm.at[0], semO[b]).wait()

    def _compute(b):
        srows_b, drows_b = srows.at[b], drows.at[b]

        for g in range(_K // 16):
            eids = lax.iota(jnp.int32, 16) + g * 16
            a1 = jnp.zeros((16,), jnp.float32)
            a2 = jnp.zeros((16,), jnp.float32)
            for j in range(EMB_DIM):
                jj = jnp.full((16,), j, jnp.int32)
                s = plsc.load_gather(srows_b, [eids, jj])
                d = plsc.load_gather(drows_b, [eids, jj])
                a1 = a1 + s * d
                a2 = a2 + s * gbv[j]
            deg = degb[b, pl.ds(g * 16, 16)]
            att = 1.0 / ((1.0 + jnp.exp(-a1)) * deg)
            r = 1.0 / (1.0 + jnp.exp(-(att * a2))) - 0.5
            outb[b, pl.ds(g * 16, 16)] = r

    def _step(t, b, with_out_wait):
        ob = 1 - b
        _wait_idx(ob)
        _issue_gathers(ob)
        _wait_gathers(b)
        _issue_idx(t + 2, b)
        pltpu.sync_copy(hist.at[didx.at[b]], degb.at[b])
        if with_out_wait:
            _wait_out(b)
        _compute(b)
        _issue_out(t, b)

    _issue_idx(0, 0)
    _issue_idx(1, 1)
    _wait_idx(0)
    _issue_gathers(0)
    _step(0, 0, False)
    _step(1, 1, False)

    def _pair(g, carry):
        _step(2 * g + 2, 0, True)
        _step(2 * g + 3, 1, True)
        return carry
    lax.fori_loop(0, (_TSTEPS - 2) // 2, _pair, 0)

    _wait_gathers(0)
    _wait_idx(1)
    _wait_out(0)
    _wait_out(1)


def kernel(user_emb, item_emb, u_trust, guided_item):
    src = u_trust[0].astype(jnp.int32)
    dst = u_trust[1].astype(jnp.int32)
    pad = _ROWS_ALL * _K - NUM_EDGES
    src2d = jnp.concatenate(
        [src, jnp.zeros((pad,), jnp.int32)]).reshape(_ROWS_ALL, _K)
    dst2d = jnp.concatenate(
        [dst, jnp.full((pad,), _PAD_BIN, jnp.int32)]).reshape(_ROWS_ALL, _K)
    g_row = lax.dynamic_index_in_dim(item_emb, guided_item, axis=0,
                                     keepdims=False)
    gb = jnp.broadcast_to(g_row[:, None], (EMB_DIM, 16))

    mesh = plsc.VectorSubcoreMesh(core_axis_name="c", subcore_axis_name="s")
    run = functools.partial(
        pl.kernel,
        mesh=mesh,
        compiler_params=pltpu.CompilerParams(needs_layout_passes=False,
                                             use_tc_tiling_on_sc=False),
        out_type=jax.ShapeDtypeStruct((_ROWS_OUT, _K), jnp.float32),
        scratch_types=[
            pltpu.VMEM_SHARED((_HIST,), jnp.float32),
            pltpu.VMEM((_HSLICE,), jnp.float32),
            pltpu.VMEM((_HC, _K), jnp.float32),
            pltpu.VMEM((2, _HC, _K), jnp.int32),
            pltpu.VMEM((2, _K), jnp.int32),
            pltpu.VMEM((2, _K), jnp.int32),
            pltpu.VMEM((2, _K, EMB_DIM), jnp.float32),
            pltpu.VMEM((2, _K, EMB_DIM), jnp.float32),
            pltpu.VMEM((2, _K), jnp.float32),
            pltpu.VMEM((2, _K), jnp.float32),
            pltpu.VMEM((EMB_DIM, 16), jnp.float32),
        ] + [pltpu.SemaphoreType.DMA] * 8,
    )(_sc_body)
    out = run(user_emb, src2d, dst2d, gb)
    return out.reshape(-1)[:NUM_EDGES].reshape(NUM_EDGES, 1)


# speedup vs baseline: 2.6457x; 2.6457x over previous
import functools

import jax
import jax.numpy as jnp
from jax import lax
from jax.experimental import pallas as pl
from jax.experimental.pallas import tpu as pltpu
from jax.experimental.pallas import tpu_sc as plsc

NUM_USERS = 100000
EMB_DIM = 32
NUM_EDGES = 1600000

_NC = 2
_NS = 16
_NW = _NC * _NS

_HIST = 100352
_HSLICE = _HIST // _NS
_PAD_BIN = _HIST - 1

_K = 128
_ROWS = NUM_EDGES // _K
_TSTEPS = 392
_ROWS_OUT = _NW * _TSTEPS
_ROWS_ALL = _ROWS_OUT + 68

_HROWS = 12608
_HPT = _HROWS // _NS
_HC = 4
_HITER = _HPT // _HC

_MASK_HI = -65536


def _tc_body(u_ref, g_ref, p_ref):
    p_ref[...] = jnp.sum(u_ref[...] * g_ref[...], axis=1, keepdims=True)


def _sc_body(user16, src2d, dst2d, p_hbm, out_hbm,
             hist, zbuf, ones4, hidx, sidx, didx, srows, drows,
             degb, pb, outb,
             semA0, semA1, semB0, semB1, semO0, semO1, semH0, semH1):
    cid = lax.axis_index("c")
    sid = lax.axis_index("s")
    wid = sid * _NC + cid
    semA = (semA0, semA1)
    semB = (semB0, semB1)
    semO = (semO0, semO1)
    semH = (semH0, semH1)

    def _zero(i, carry):
        zbuf[pl.ds(i * 16, 16)] = jnp.zeros((16,), jnp.float32)
        return carry
    lax.fori_loop(0, _HSLICE // 16, _zero, 0)
    for r in range(_HC):
        for i in range(_K // 16):
            ones4[r, pl.ds(i * 16, 16)] = jnp.ones((16,), jnp.float32)
    pltpu.sync_copy(zbuf, hist.at[pl.ds(pl.multiple_of(sid * _HSLICE, 8),
                                        _HSLICE)])
    plsc.subcore_barrier()

    hrow0 = sid * _HPT

    def _h_load(k, b):
        pltpu.make_async_copy(dst2d.at[pl.ds(hrow0 + k * _HC, _HC), :],
                              hidx.at[b], semH[b]).start()

    def _h_wait(b):
        pltpu.make_async_copy(dst2d.at[pl.ds(0, _HC), :],
                              hidx.at[b], semH[b]).wait()

    def _h_scatter(b):
        for j in range(_HC):
            pltpu.sync_copy(ones4.at[0], hist.at[hidx.at[b, j]], add=True)

    _h_load(0, 0)
    _h_wait(0)
    _h_load(1, 1)
    _h_scatter(0)

    def _h_pair(g, carry):
        k = 2 * g + 1
        _h_wait(1)
        _h_load(k + 1, 0)
        _h_scatter(1)
        _h_wait(0)
        _h_load(k + 2, 1)
        _h_scatter(0)
        return carry
    lax.fori_loop(0, (_HITER - 1) // 2, _h_pair, 0)
    _h_wait(1)
    plsc.subcore_barrier()

    def _issue_idx(t, b):
        c = wid + t * _NW
        pltpu.make_async_copy(src2d.at[c], sidx.at[b], semA[b]).start()
        pltpu.make_async_copy(dst2d.at[c], didx.at[b], semA[b]).start()

    def _wait_idx(b):
        pltpu.make_async_copy(src2d.at[0], sidx.at[b], semA[b]).wait()
        pltpu.make_async_copy(dst2d.at[0], didx.at[b], semA[b]).wait()

    def _issue_gathers(b):
        pltpu.make_async_copy(user16.at[sidx.at[b]], srows.at[b],
                              semB[b]).start()
        pltpu.make_async_copy(user16.at[didx.at[b]], drows.at[b],
                              semB[b]).start()
        pltpu.make_async_copy(p_hbm.at[sidx.at[b]], pb.at[b],
                              semB[b]).start()

    def _wait_gathers(b):
        pltpu.make_async_copy(user16.at[pl.ds(0, _K), :], srows.at[b],
                              semB[b]).wait()
        pltpu.make_async_copy(user16.at[pl.ds(0, _K), :], drows.at[b],
                              semB[b]).wait()
        pltpu.make_async_copy(p_hbm.at[pl.ds(0, _K)], pb.at[b],
                              semB[b]).wait()

    def _issue_out(t, b):
        c = wid + t * _NW
        pltpu.make_async_copy(outb.at[b], out_hbm.at[c], semO[b]).start()

    def _wait_out(b):
        pltpu.make_async_copy(outb.at[b], out_hbm.at[0], semO[b]).wait()

    def _compute(b):
        srows_i = srows.at[b]
        drows_i = drows.at[b]

        def _grp(g, c2):
            gb16 = pl.multiple_of(g * 16, 8)
            eids = lax.iota(jnp.int32, 16) + g * 16
            a1 = jnp.zeros((16,), jnp.float32)
            for j in range(EMB_DIM // 2):
                jj = jnp.full((16,), j, jnp.int32)
                sp = plsc.load_gather(srows_i, [eids, jj])
                dp = plsc.load_gather(drows_i, [eids, jj])
                slo = plsc.bitcast(sp << 16, jnp.float32)
                shi = plsc.bitcast(sp & _MASK_HI, jnp.float32)
                dlo = plsc.bitcast(dp << 16, jnp.float32)
                dhi = plsc.bitcast(dp & _MASK_HI, jnp.float32)
                a1 = a1 + slo * dlo + shi * dhi
            deg = degb[b, pl.ds(gb16, 16)]
            a2 = pb[b, pl.ds(gb16, 16)]
            att = 1.0 / ((1.0 + jnp.exp(-a1)) * deg)
            r = 1.0 / (1.0 + jnp.exp(-(att * a2))) - 0.5
            outb[b, pl.ds(gb16, 16)] = r
            return c2
        lax.fori_loop(0, _K // 16, _grp, 0)

    def _step(t, b, with_out_wait):
        ob = 1 - b
        _wait_idx(ob)
        _issue_gathers(ob)
        _wait_gathers(b)
        _issue_idx(t + 2, b)
        pltpu.sync_copy(hist.at[didx.at[b]], degb.at[b])
        if with_out_wait:
            _wait_out(b)
        _compute(b)
        _issue_out(t, b)

    _issue_idx(0, 0)
    _issue_idx(1, 1)
    _wait_idx(0)
    _issue_gathers(0)
    _step(0, 0, False)
    _step(1, 1, False)

    def _pair(g, carry):
        _step(2 * g + 2, 0, True)
        _step(2 * g + 3, 1, True)
        return carry
    lax.fori_loop(0, (_TSTEPS - 2) // 2, _pair, 0)

    _wait_gathers(0)
    _wait_idx(1)
    _wait_out(0)
    _wait_out(1)


def kernel(user_emb, item_emb, u_trust, guided_item):
    src = u_trust[0].astype(jnp.int32)
    dst = u_trust[1].astype(jnp.int32)
    pad = _ROWS_ALL * _K - NUM_EDGES
    src2d = jnp.concatenate(
        [src, jnp.zeros((pad,), jnp.int32)]).reshape(_ROWS_ALL, _K)
    dst2d = jnp.concatenate(
        [dst, jnp.full((pad,), _PAD_BIN, jnp.int32)]).reshape(_ROWS_ALL, _K)
    g_row = lax.dynamic_index_in_dim(item_emb, guided_item, axis=0,
                                     keepdims=True)
    user16 = lax.bitcast_convert_type(
        user_emb.astype(jnp.bfloat16).reshape(NUM_USERS, EMB_DIM // 2, 2),
        jnp.int32)

    p = pl.pallas_call(
        _tc_body,
        grid=(10,),
        in_specs=[
            pl.BlockSpec((NUM_USERS // 10, EMB_DIM), lambda i: (i, 0)),
            pl.BlockSpec((1, EMB_DIM), lambda i: (0, 0)),
        ],
        out_specs=pl.BlockSpec((NUM_USERS // 10, 1), lambda i: (i, 0)),
        out_shape=jax.ShapeDtypeStruct((NUM_USERS, 1), jnp.float32),
    )(user_emb, g_row).reshape(NUM_USERS)

    mesh = plsc.VectorSubcoreMesh(core_axis_name="c", subcore_axis_name="s")
    run = functools.partial(
        pl.kernel,
        mesh=mesh,
        compiler_params=pltpu.CompilerParams(needs_layout_passes=False,
                                             use_tc_tiling_on_sc=False),
        out_type=jax.ShapeDtypeStruct((_ROWS_OUT, _K), jnp.float32),
        scratch_types=[
            pltpu.VMEM_SHARED((_HIST,), jnp.float32),
            pltpu.VMEM((_HSLICE,), jnp.float32),
            pltpu.VMEM((_HC, _K), jnp.float32),
            pltpu.VMEM((2, _HC, _K), jnp.int32),
            pltpu.VMEM((2, _K), jnp.int32),
            pltpu.VMEM((2, _K), jnp.int32),
            pltpu.VMEM((2, _K, EMB_DIM // 2), jnp.int32),
            pltpu.VMEM((2, _K, EMB_DIM // 2), jnp.int32),
            pltpu.VMEM((2, _K), jnp.float32),
            pltpu.VMEM((2, _K), jnp.float32),
            pltpu.VMEM((2, _K), jnp.float32),
        ] + [pltpu.SemaphoreType.DMA] * 8,
    )(_sc_body)
    out = run(user16, src2d, dst2d, p)
    return out.reshape(-1)[:NUM_EDGES].reshape(NUM_EDGES, 1)
